# baseline (speedup 1.0000x reference)
import functools

import jax
import jax.numpy as jnp
from jax import lax
from jax.experimental import pallas as pl
from jax.experimental.pallas import tpu as pltpu
from jax.experimental.pallas import tpu_sc as plsc

_B = 16384
_N = 8
_NP = 16
_G = 128
_D = _N * _G
_DS = 512

_NC = 2
_NS = 16
_NW = _NC * _NS
_BPW = _B // _NW
_CH = 32
_NCHUNK = _BPW // _CH


def _gates_body(state_ref, w_ref, b_ref, out_ref):
    logits = jnp.dot(state_ref[...], w_ref[...],
                     preferred_element_type=jnp.float32) + b_ref[...]
    m = jnp.max(logits, axis=-1, keepdims=True)
    e = jnp.exp(logits - m)
    out_ref[...] = e / jnp.sum(e, axis=-1, keepdims=True)


def _gates_tc(state, w, b):
    blk = 2048
    grid = _B // blk
    return pl.pallas_call(
        _gates_body,
        grid=(grid,),
        in_specs=[
            pl.BlockSpec((blk, _DS), lambda i: (i, 0)),
            pl.BlockSpec((_DS, _NP), lambda i: (0, 0)),
            pl.BlockSpec((1, _NP), lambda i: (0, 0)),
        ],
        out_specs=pl.BlockSpec((blk, _NP), lambda i: (i, 0)),
        out_shape=jax.ShapeDtypeStruct((_B, _NP), jnp.float32),
    )(state, w, b.reshape(1, _NP))


@functools.cache
def _make_sc_combine():
    mesh = plsc.VectorSubcoreMesh(core_axis_name="c", subcore_axis_name="s")
    return pl.kernel(
        _sc_combine_body,
        mesh=mesh,
        out_type=jax.ShapeDtypeStruct((_B, _G), jnp.float32),
        scratch_types=[
            pltpu.VMEM((_NCHUNK, _CH), jnp.int32),
            pltpu.VMEM((_CH, _NP), jnp.float32),
            pltpu.VMEM((_CH, _D), jnp.float32),
            pltpu.VMEM((_CH, _G), jnp.float32),
            pltpu.SemaphoreType.DMA,
        ],
    )


def _sc_combine_body(idx_hbm, gates_hbm, table_hbm, out_hbm,
                     idx_v, gates_v, rows_v, out_v, sem):
    wid = lax.axis_index("s") * _NC + lax.axis_index("c")
    base = wid * _BPW
    pltpu.sync_copy(idx_hbm.at[pl.ds(wid * _NCHUNK, _NCHUNK)], idx_v)

    def chunk_body(c, carry):
        pltpu.sync_copy(gates_hbm.at[pl.ds(base + c * _CH, _CH)], gates_v)
        pltpu.async_copy(table_hbm.at[idx_v.at[c]], rows_v, sem).wait()

        def row_body(r, carry2):
            gv = gates_v[r, pl.ds(0, _NP)]
            gs = [gv[n] for n in range(_N)]
            for j in range(_G // 16):
                acc = rows_v[r, pl.ds(j * 16, 16)] * gs[0]
                for n in range(1, _N):
                    acc = acc + rows_v[r, pl.ds(n * _G + j * 16, 16)] * gs[n]
                out_v[r, pl.ds(j * 16, 16)] = acc
            return carry2

        lax.fori_loop(0, _CH, row_body, 0)
        pltpu.sync_copy(out_v, out_hbm.at[pl.ds(base + c * _CH, _CH)])
        return carry

    lax.fori_loop(0, _NCHUNK, chunk_body, 0)


def kernel(latent_id, state, latents, W_gate, b_gate):
    idx2d = latent_id.astype(jnp.int32).reshape(_B // _CH, _CH)
    w_pad = jnp.pad(W_gate.astype(jnp.float32), ((0, 0), (0, _NP - _N)))
    b_pad = jnp.pad(b_gate.astype(jnp.float32), (0, _NP - _N),
                    constant_values=-1e30)
    gates = _gates_tc(state, w_pad, b_pad)
    table = latents.reshape(latents.shape[0], _D)
    return _make_sc_combine()(idx2d, gates, table)


# speedup vs baseline: 2.9455x; 2.9455x over previous
import functools

import jax
import jax.numpy as jnp
from jax import lax
from jax.experimental import pallas as pl
from jax.experimental.pallas import tpu as pltpu
from jax.experimental.pallas import tpu_sc as plsc

_B = 16384
_N = 8
_NP = 16
_G = 128
_D = _N * _G
_DS = 512

_NC = 2
_NS = 16
_NW = _NC * _NS
_BPW = _B // _NW
_CH = 32
_NCHUNK = _BPW // _CH


def _gates_body(state_ref, w_ref, b_ref, out_ref):
    logits = jnp.dot(state_ref[...], w_ref[...],
                     preferred_element_type=jnp.float32) + b_ref[...]
    m = jnp.max(logits, axis=-1, keepdims=True)
    e = jnp.exp(logits - m)
    out_ref[...] = e / jnp.sum(e, axis=-1, keepdims=True)


def _gates_tc(state, w, b):
    blk = 2048
    grid = _B // blk
    return pl.pallas_call(
        _gates_body,
        grid=(grid,),
        in_specs=[
            pl.BlockSpec((blk, _DS), lambda i: (i, 0)),
            pl.BlockSpec((_DS, _NP), lambda i: (0, 0)),
            pl.BlockSpec((1, _NP), lambda i: (0, 0)),
        ],
        out_specs=pl.BlockSpec((blk, _NP), lambda i: (i, 0)),
        out_shape=jax.ShapeDtypeStruct((_B, _NP), jnp.float32),
    )(state, w, b.reshape(1, _NP))


@functools.cache
def _make_sc_combine():
    mesh = plsc.VectorSubcoreMesh(core_axis_name="c", subcore_axis_name="s")
    return pl.kernel(
        _sc_combine_body,
        mesh=mesh,
        out_type=jax.ShapeDtypeStruct((_B, _G), jnp.float32),
        scratch_types=[
            pltpu.VMEM((_NCHUNK, _CH), jnp.int32),
            pltpu.VMEM((_CH, _NP), jnp.float32),
            pltpu.VMEM((_CH, _N, _G), jnp.float32),
            pltpu.VMEM((_CH, _G), jnp.float32),
            pltpu.SemaphoreType.DMA,
        ],
    )


def _sc_combine_body(idx_hbm, gates_hbm, table_hbm, out_hbm,
                     idx_v, gates_v, rows_v, out_v, sem):
    wid = lax.axis_index("s") * _NC + lax.axis_index("c")
    base = wid * _BPW
    pltpu.sync_copy(idx_hbm.at[pl.ds(wid * _NCHUNK, _NCHUNK)], idx_v)

    def chunk_body(c, carry):
        pltpu.sync_copy(gates_hbm.at[pl.ds(base + c * _CH, _CH)], gates_v)
        pltpu.async_copy(table_hbm.at[idx_v.at[c]], rows_v, sem).wait()

        def row_body(r, carry2):
            gv = gates_v[r, pl.ds(0, _NP)]
            gs = [gv[n] for n in range(_N)]
            for j in range(_G // 16):
                acc = rows_v[r, 0, pl.ds(j * 16, 16)] * gs[0]
                for n in range(1, _N):
                    acc = acc + rows_v[r, n, pl.ds(j * 16, 16)] * gs[n]
                out_v[r, pl.ds(j * 16, 16)] = acc
            return carry2

        lax.fori_loop(0, _CH, row_body, 0)
        pltpu.sync_copy(out_v, out_hbm.at[pl.ds(base + c * _CH, _CH)])
        return carry

    lax.fori_loop(0, _NCHUNK, chunk_body, 0)


def kernel(latent_id, state, latents, W_gate, b_gate):
    idx2d = latent_id.astype(jnp.int32).reshape(_B // _CH, _CH)
    w_pad = jnp.pad(W_gate.astype(jnp.float32), ((0, 0), (0, _NP - _N)))
    b_pad = jnp.pad(b_gate.astype(jnp.float32), (0, _NP - _N),
                    constant_values=-1e30)
    gates = _gates_tc(state, w_pad, b_pad)
    return _make_sc_combine()(idx2d, gates, latents)


# speedup vs baseline: 4.0360x; 1.3702x over previous
import functools

import jax
import jax.numpy as jnp
from jax import lax
from jax.experimental import pallas as pl
from jax.experimental.pallas import tpu as pltpu
from jax.experimental.pallas import tpu_sc as plsc

_B = 16384
_N = 8
_NP = 16
_G = 128
_D = _N * _G
_DS = 512

_NC = 2
_NS = 16
_NW = _NC * _NS
_BPW = _B // _NW
_CH = 16
_NCHUNK = _BPW // _CH


def _gates_body(state_ref, w_ref, b_ref, out_ref):
    logits = jnp.dot(state_ref[...], w_ref[...],
                     preferred_element_type=jnp.float32) + b_ref[...]
    m = jnp.max(logits, axis=-1, keepdims=True)
    e = jnp.exp(logits - m)
    out_ref[...] = e / jnp.sum(e, axis=-1, keepdims=True)


def _gates_tc(state, w, b):
    blk = 2048
    grid = _B // blk
    return pl.pallas_call(
        _gates_body,
        grid=(grid,),
        in_specs=[
            pl.BlockSpec((blk, _DS), lambda i: (i, 0)),
            pl.BlockSpec((_DS, _NP), lambda i: (0, 0)),
            pl.BlockSpec((1, _NP), lambda i: (0, 0)),
        ],
        out_specs=pl.BlockSpec((blk, _NP), lambda i: (i, 0)),
        out_shape=jax.ShapeDtypeStruct((_B, _NP), jnp.float32),
    )(state, w, b.reshape(1, _NP))


@functools.cache
def _make_sc_combine():
    mesh = plsc.VectorSubcoreMesh(core_axis_name="c", subcore_axis_name="s")
    return pl.kernel(
        _sc_combine_body,
        mesh=mesh,
        out_type=jax.ShapeDtypeStruct((_B, _G), jnp.float32),
        scratch_types=[
            pltpu.VMEM((_NCHUNK, _CH), jnp.int32),
            pltpu.VMEM((_BPW, _NP), jnp.float32),
            pltpu.VMEM((_CH, _N, _G), jnp.float32),
            pltpu.VMEM((_CH, _N, _G), jnp.float32),
            pltpu.VMEM((_CH, _G), jnp.float32),
            pltpu.SemaphoreType.DMA,
            pltpu.SemaphoreType.DMA,
        ],
    )


def _sc_combine_body(idx_hbm, gates_hbm, table_hbm, out_hbm,
                     idx_v, gates_v, rows_a, rows_b, out_v, sem_a, sem_b):
    wid = lax.axis_index("s") * _NC + lax.axis_index("c")
    base = wid * _BPW
    pltpu.sync_copy(idx_hbm.at[pl.ds(wid * _NCHUNK, _NCHUNK)], idx_v)
    pltpu.sync_copy(gates_hbm.at[pl.ds(base, _BPW)], gates_v)

    def combine_chunk(c, rows_v):
        def row_body(r, carry):
            gv = gates_v[c * _CH + r, pl.ds(0, _NP)]
            gs = [gv[n] for n in range(_N)]
            for j in range(_G // 16):
                acc = rows_v[r, 0, pl.ds(j * 16, 16)] * gs[0]
                for n in range(1, _N):
                    acc = acc + rows_v[r, n, pl.ds(j * 16, 16)] * gs[n]
                out_v[r, pl.ds(j * 16, 16)] = acc
            return carry

        lax.fori_loop(0, _CH, row_body, 0)
        pltpu.sync_copy(out_v, out_hbm.at[pl.ds(base + c * _CH, _CH)])

    def gather(c, rows_v, sem):
        pltpu.async_copy(table_hbm.at[idx_v.at[c]], rows_v, sem)

    def gather_wait(c, rows_v, sem):
        pltpu.make_async_copy(table_hbm.at[idx_v.at[c]], rows_v, sem).wait()

    gather(0, rows_a, sem_a)

    def pair_body(p, carry):
        c = p * 2
        gather(c + 1, rows_b, sem_b)
        gather_wait(c, rows_a, sem_a)
        combine_chunk(c, rows_a)

        @pl.when(p < _NCHUNK // 2 - 1)
        def _():
            gather(c + 2, rows_a, sem_a)

        gather_wait(c + 1, rows_b, sem_b)
        combine_chunk(c + 1, rows_b)
        return carry

    lax.fori_loop(0, _NCHUNK // 2, pair_body, 0)


def kernel(latent_id, state, latents, W_gate, b_gate):
    idx2d = latent_id.astype(jnp.int32).reshape(_B // _CH, _CH)
    w_pad = jnp.pad(W_gate.astype(jnp.float32), ((0, 0), (0, _NP - _N)))
    b_pad = jnp.pad(b_gate.astype(jnp.float32), (0, _NP - _N),
                    constant_values=-1e30)
    gates = _gates_tc(state, w_pad, b_pad)
    return _make_sc_combine()(idx2d, gates, latents)


# speedup vs baseline: 4.1803x; 1.0358x over previous
import functools

import jax
import jax.numpy as jnp
from jax import lax
from jax.experimental import pallas as pl
from jax.experimental.pallas import tpu as pltpu
from jax.experimental.pallas import tpu_sc as plsc

_B = 16384
_N = 8
_NP = 16
_G = 128
_D = _N * _G
_DS = 512

_NC = 2
_NS = 16
_NW = _NC * _NS
_BPW = _B // _NW
_CH = 16
_NCHUNK = _BPW // _CH


def _gates_body(state_ref, w_ref, b_ref, out_ref):
    logits = jnp.dot(state_ref[...], w_ref[...],
                     preferred_element_type=jnp.float32) + b_ref[...]
    m = jnp.max(logits, axis=-1, keepdims=True)
    e = jnp.exp(logits - m)
    out_ref[...] = e / jnp.sum(e, axis=-1, keepdims=True)


def _gates_tc(state, w, b):
    blk = 2048
    grid = _B // blk
    return pl.pallas_call(
        _gates_body,
        grid=(grid,),
        in_specs=[
            pl.BlockSpec((blk, _DS), lambda i: (i, 0)),
            pl.BlockSpec((_DS, _NP), lambda i: (0, 0)),
            pl.BlockSpec((1, _NP), lambda i: (0, 0)),
        ],
        out_specs=pl.BlockSpec((blk, _NP), lambda i: (i, 0)),
        out_shape=jax.ShapeDtypeStruct((_B, _NP), jnp.float32),
    )(state, w, b.reshape(1, _NP))


@functools.cache
def _make_sc_combine():
    mesh = plsc.VectorSubcoreMesh(core_axis_name="c", subcore_axis_name="s")
    return pl.kernel(
        _sc_combine_body,
        mesh=mesh,
        out_type=jax.ShapeDtypeStruct((_B, _G), jnp.float32),
        scratch_types=[
            pltpu.VMEM((_NCHUNK, _CH), jnp.int32),
            pltpu.VMEM((_BPW, _NP), jnp.float32),
            pltpu.VMEM((_CH, _N, _G), jnp.float32),
            pltpu.VMEM((_CH, _N, _G), jnp.float32),
            pltpu.VMEM((_CH, _G), jnp.float32),
            pltpu.VMEM((_CH, _G), jnp.float32),
            pltpu.SemaphoreType.DMA,
            pltpu.SemaphoreType.DMA,
            pltpu.SemaphoreType.DMA,
            pltpu.SemaphoreType.DMA,
        ],
    )


def _sc_combine_body(idx_hbm, gates_hbm, table_hbm, out_hbm,
                     idx_v, gates_v, rows_a, rows_b, out_a, out_b,
                     sem_a, sem_b, sem_oa, sem_ob):
    wid = lax.axis_index("s") * _NC + lax.axis_index("c")
    base = wid * _BPW
    pltpu.sync_copy(idx_hbm.at[pl.ds(wid * _NCHUNK, _NCHUNK)], idx_v)
    pltpu.sync_copy(gates_hbm.at[pl.ds(base, _BPW)], gates_v)

    def combine_chunk(c, rows_v, out_v):
        def row_body(r, carry):
            gv = gates_v[c * _CH + r, pl.ds(0, _NP)]
            gs = [gv[n] for n in range(_N)]
            for j in range(_G // 16):
                acc = rows_v[r, 0, pl.ds(j * 16, 16)] * gs[0]
                for n in range(1, _N):
                    acc = acc + rows_v[r, n, pl.ds(j * 16, 16)] * gs[n]
                out_v[r, pl.ds(j * 16, 16)] = acc
            return carry

        lax.fori_loop(0, _CH, row_body, 0, unroll=2)

    def gather(c, rows_v, sem):
        pltpu.async_copy(table_hbm.at[idx_v.at[c]], rows_v, sem)

    def gather_wait(c, rows_v, sem):
        pltpu.make_async_copy(table_hbm.at[idx_v.at[c]], rows_v, sem).wait()

    def out_slice(c):
        return out_hbm.at[pl.ds(base + c * _CH, _CH)]

    gather(0, rows_a, sem_a)

    def pair_body(p, carry):
        c = p * 2
        gather(c + 1, rows_b, sem_b)
        gather_wait(c, rows_a, sem_a)

        @pl.when(p > 0)
        def _():
            pltpu.make_async_copy(out_a, out_slice(c), sem_oa).wait()

        combine_chunk(c, rows_a, out_a)
        pltpu.async_copy(out_a, out_slice(c), sem_oa)

        @pl.when(p < _NCHUNK // 2 - 1)
        def _():
            gather(c + 2, rows_a, sem_a)

        gather_wait(c + 1, rows_b, sem_b)

        @pl.when(p > 0)
        def _():
            pltpu.make_async_copy(out_b, out_slice(c + 1), sem_ob).wait()

        combine_chunk(c + 1, rows_b, out_b)
        pltpu.async_copy(out_b, out_slice(c + 1), sem_ob)
        return carry

    lax.fori_loop(0, _NCHUNK // 2, pair_body, 0)
    pltpu.make_async_copy(out_a, out_slice(0), sem_oa).wait()
    pltpu.make_async_copy(out_b, out_slice(1), sem_ob).wait()


def kernel(latent_id, state, latents, W_gate, b_gate):
    idx2d = latent_id.astype(jnp.int32).reshape(_B // _CH, _CH)
    w_pad = jnp.pad(W_gate.astype(jnp.float32), ((0, 0), (0, _NP - _N)))
    b_pad = jnp.pad(b_gate.astype(jnp.float32), (0, _NP - _N),
                    constant_values=-1e30)
    gates = _gates_tc(state, w_pad, b_pad)
    return _make_sc_combine()(idx2d, gates, latents)
